# baseline (speedup 1.0000x reference)
import functools

import jax
import jax.numpy as jnp
from jax import lax
from jax.experimental import pallas as pl
from jax.experimental.pallas import tpu as pltpu
from jax.experimental.pallas import tpu_sc as plsc

E = 8
D = 1024
DF = 1024
T = 4096

BLK = 128
NBLK = 2 * T // BLK + E
NSLOT = NBLK * BLK

NW = 32
TPW = T // NW
CH = 32
NCH = TPW // CH

BTC = 256
H = D // 2


def _pack2(v):
    u = jax.lax.bitcast_convert_type(v, jnp.uint32)
    u = (u + jnp.uint32(0x8000)) >> 16
    a = u[:, :H]
    b = u[:, H:]
    return jax.lax.bitcast_convert_type((b << 16) | a, jnp.int32)


def _unpack2(p):
    u = jax.lax.bitcast_convert_type(p, jnp.uint32)
    lo = jax.lax.bitcast_convert_type(u << 16, jnp.float32)
    hi = jax.lax.bitcast_convert_type(u & jnp.uint32(0xFFFF0000), jnp.float32)
    return lo, hi


def _route_kernel(x_ref, wg_ref, pos0_ref, pos1_ref, g0_ref, g1_ref, be_ref,
                  xpk_ref):
    xb = x_ref[...]
    xpk_ref[...] = _pack2(xb)
    logits = jnp.dot(xb, wg_ref[...].T, preferred_element_type=jnp.float32)
    z = logits - jnp.max(logits, axis=-1, keepdims=True)
    ez = jnp.exp(z)
    sc = ez / jnp.sum(ez, axis=-1, keepdims=True)

    lane = lax.broadcasted_iota(jnp.int32, sc.shape, 1)
    m1 = jnp.max(sc, axis=-1, keepdims=True)
    i1 = jnp.min(jnp.where(sc == m1, lane, E), axis=-1, keepdims=True)
    s2 = jnp.where(lane == i1, -jnp.inf, sc)
    m2 = jnp.max(s2, axis=-1, keepdims=True)
    i2 = jnp.min(jnp.where(s2 == m2, lane, E), axis=-1, keepdims=True)

    C1 = (lane == i1).astype(jnp.float32)
    C2 = (lane == i2).astype(jnp.float32)

    R = 512
    G = T // R
    ri = lax.broadcasted_iota(jnp.int32, (R, R), 0)
    ci = lax.broadcasted_iota(jnp.int32, (R, R), 1)
    trilx = (ci < ri).astype(jnp.float32)
    ones_row = jnp.ones((1, R), jnp.float32)

    p1, p2, s1, s2l = [], [], [], []
    for g in range(G):
        cg1 = C1[g * R:(g + 1) * R, :]
        cg2 = C2[g * R:(g + 1) * R, :]
        p1.append(jnp.dot(trilx, cg1, preferred_element_type=jnp.float32))
        p2.append(jnp.dot(trilx, cg2, preferred_element_type=jnp.float32))
        s1.append(jnp.dot(ones_row, cg1, preferred_element_type=jnp.float32))
        s2l.append(jnp.dot(ones_row, cg2, preferred_element_type=jnp.float32))
    S1 = jnp.concatenate(s1, axis=0)
    S2 = jnp.concatenate(s2l, axis=0)
    gr = lax.broadcasted_iota(jnp.int32, (G, G), 0)
    gc = lax.broadcasted_iota(jnp.int32, (G, G), 1)
    trilg = (gc < gr).astype(jnp.float32)
    O1 = jnp.dot(trilg, S1, preferred_element_type=jnp.float32)
    O2 = jnp.dot(trilg, S2, preferred_element_type=jnp.float32)
    prefix1 = jnp.concatenate([p1[g] + O1[g:g + 1, :] for g in range(G)], 0)
    prefix2 = jnp.concatenate([p2[g] + O2[g:g + 1, :] for g in range(G)], 0)
    tot1 = jnp.sum(S1, axis=0, keepdims=True)
    tot2 = jnp.sum(S2, axis=0, keepdims=True)

    counts = tot1 + tot2
    nb = jnp.ceil(counts / BLK)
    er = lax.broadcasted_iota(jnp.int32, (E, E), 0)
    ec = lax.broadcasted_iota(jnp.int32, (E, E), 1)
    upx = (er < ec).astype(jnp.float32)
    slot_base = BLK * jnp.dot(nb, upx, preferred_element_type=jnp.float32)

    rank1 = jnp.sum(prefix1 * C1, axis=1, keepdims=True)
    base1 = jnp.sum(slot_base * C1, axis=1, keepdims=True)
    rank2 = jnp.sum((tot1 + prefix2) * C2, axis=1, keepdims=True)
    base2 = jnp.sum(slot_base * C2, axis=1, keepdims=True)
    pos0_ref[...] = (base1 + rank1).astype(jnp.int32)
    pos1_ref[...] = (base2 + rank2).astype(jnp.int32)
    g0_ref[...] = m1
    g1_ref[...] = m2

    cum_incl = jnp.sum((ec <= er).astype(jnp.float32) * nb, axis=1,
                       keepdims=True).astype(jnp.int32)
    bi = lax.broadcasted_iota(jnp.int32, (E, NBLK), 1)
    be = jnp.sum((bi >= cum_incl).astype(jnp.int32), axis=0, keepdims=True)
    be_ref[...] = be


def _route(x, Wg):
    full = lambda shape: pl.BlockSpec(shape, lambda: (0,) * len(shape))
    return pl.pallas_call(
        _route_kernel,
        in_specs=[full((T, D)), full((E, D))],
        out_specs=[full((T, 1)), full((T, 1)), full((T, 1)), full((T, 1)),
                   full((1, NBLK)), full((T, H))],
        out_shape=[
            jax.ShapeDtypeStruct((T, 1), jnp.int32),
            jax.ShapeDtypeStruct((T, 1), jnp.int32),
            jax.ShapeDtypeStruct((T, 1), jnp.float32),
            jax.ShapeDtypeStruct((T, 1), jnp.float32),
            jax.ShapeDtypeStruct((1, NBLK), jnp.int32),
            jax.ShapeDtypeStruct((T, H), jnp.int32),
        ],
    )(x, Wg)


@functools.cache
def _sc_mesh():
    return plsc.VectorSubcoreMesh(core_axis_name="c", subcore_axis_name="s")


@functools.cache
def _sc_dispatch_kernel():
    @functools.partial(
        pl.kernel,
        out_type=jax.ShapeDtypeStruct((NSLOT, H), jnp.int32),
        mesh=_sc_mesh(),
        scratch_types=[
            pltpu.VMEM((NCH, CH), jnp.int32),
            pltpu.VMEM((NCH, CH), jnp.int32),
            pltpu.VMEM((CH, H), jnp.int32),
            pltpu.SemaphoreType.DMA,
        ],
    )
    def body(x_hbm, pos0_hbm, pos1_hbm, xg_hbm, idx0_v, idx1_v, xbuf_v, sem):
        wid = lax.axis_index("s") * 2 + lax.axis_index("c")
        base = wid * TPW
        pltpu.sync_copy(pos0_hbm.at[wid], idx0_v)
        pltpu.sync_copy(pos1_hbm.at[wid], idx1_v)
        for j in range(NCH):
            pltpu.sync_copy(x_hbm.at[pl.ds(base + j * CH, CH)], xbuf_v)
            a = pltpu.async_copy(xbuf_v, xg_hbm.at[idx0_v.at[j]], sem)
            b = pltpu.async_copy(xbuf_v, xg_hbm.at[idx1_v.at[j]], sem)
            a.wait()
            b.wait()

    return body


def _sc_dispatch(x, pos0w, pos1w):
    return _sc_dispatch_kernel()(x, pos0w, pos1w)


def _ffn_kernel(be_ref, xg_ref, w1_ref, w2_ref, w3_ref, out_ref):
    @pl.when(be_ref[pl.program_id(0)] < E)
    def _():
        lo, hi = _unpack2(xg_ref[...])
        lo = lo.astype(jnp.bfloat16)
        hi = hi.astype(jnp.bfloat16)
        w1 = w1_ref[0].astype(jnp.bfloat16)
        w3 = w3_ref[0].astype(jnp.bfloat16)
        w2 = w2_ref[0].astype(jnp.bfloat16)
        h1 = (jnp.dot(lo, w1[:, :H].T, preferred_element_type=jnp.float32)
              + jnp.dot(hi, w1[:, H:].T, preferred_element_type=jnp.float32))
        h3 = (jnp.dot(lo, w3[:, :H].T, preferred_element_type=jnp.float32)
              + jnp.dot(hi, w3[:, H:].T, preferred_element_type=jnp.float32))
        h = (jax.nn.silu(h1) * h3).astype(jnp.bfloat16)
        ye = jnp.dot(h, w2.T, preferred_element_type=jnp.float32)
        out_ref[...] = _pack2(ye)


def _ffn(be, xg, W1, W2, W3):
    wix = lambda b, be: (jnp.minimum(be[b], E - 1), 0, 0)
    grid_spec = pltpu.PrefetchScalarGridSpec(
        num_scalar_prefetch=1,
        grid=(NBLK,),
        in_specs=[
            pl.BlockSpec((BLK, H), lambda b, be: (b, 0)),
            pl.BlockSpec((1, DF, D), wix),
            pl.BlockSpec((1, D, DF), wix),
            pl.BlockSpec((1, DF, D), wix),
        ],
        out_specs=pl.BlockSpec((BLK, H), lambda b, be: (b, 0)),
    )
    return pl.pallas_call(
        _ffn_kernel,
        grid_spec=grid_spec,
        out_shape=jax.ShapeDtypeStruct((NSLOT, H), jnp.int32),
        compiler_params=pltpu.CompilerParams(
            dimension_semantics=("arbitrary",),
        ),
    )(be, xg, W1, W2, W3)


@functools.cache
def _sc_gather_kernel():
    @functools.partial(
        pl.kernel,
        out_type=[jax.ShapeDtypeStruct((T, H), jnp.int32),
                  jax.ShapeDtypeStruct((T, H), jnp.int32)],
        mesh=_sc_mesh(),
        scratch_types=[
            pltpu.VMEM((NCH, CH), jnp.int32),
            pltpu.VMEM((NCH, CH), jnp.int32),
            pltpu.VMEM((CH, H), jnp.int32),
            pltpu.VMEM((CH, H), jnp.int32),
            pltpu.SemaphoreType.DMA,
            pltpu.SemaphoreType.DMA,
        ],
    )
    def body(ys_hbm, pos0_hbm, pos1_hbm, y0_hbm, y1_hbm, idx0_v, idx1_v,
             b0_v, b1_v, s0, s1):
        wid = lax.axis_index("s") * 2 + lax.axis_index("c")
        base = wid * TPW
        pltpu.sync_copy(pos0_hbm.at[wid], idx0_v)
        pltpu.sync_copy(pos1_hbm.at[wid], idx1_v)
        for j in range(NCH):
            a = pltpu.async_copy(ys_hbm.at[idx0_v.at[j]], b0_v, s0)
            b = pltpu.async_copy(ys_hbm.at[idx1_v.at[j]], b1_v, s1)
            a.wait()
            b.wait()
            pltpu.sync_copy(b0_v, y0_hbm.at[pl.ds(base + j * CH, CH)])
            pltpu.sync_copy(b1_v, y1_hbm.at[pl.ds(base + j * CH, CH)])

    return body


def _sc_gather(ys, pos0w, pos1w):
    return _sc_gather_kernel()(ys, pos0w, pos1w)


def _combine_kernel(x_ref, y0_ref, y1_ref, g0_ref, g1_ref,
                    ws1_ref, ws2_ref, ws3_ref, out_ref):
    lo, hi = _unpack2(x_ref[...])
    lo = lo.astype(jnp.bfloat16)
    hi = hi.astype(jnp.bfloat16)
    ws1 = ws1_ref[...].astype(jnp.bfloat16)
    ws3 = ws3_ref[...].astype(jnp.bfloat16)
    ws2 = ws2_ref[...].astype(jnp.bfloat16)
    h1 = (jnp.dot(lo, ws1[:, :H].T, preferred_element_type=jnp.float32)
          + jnp.dot(hi, ws1[:, H:].T, preferred_element_type=jnp.float32))
    h3 = (jnp.dot(lo, ws3[:, :H].T, preferred_element_type=jnp.float32)
          + jnp.dot(hi, ws3[:, H:].T, preferred_element_type=jnp.float32))
    h = (jax.nn.silu(h1) * h3).astype(jnp.bfloat16)
    sh = jnp.dot(h, ws2.T, preferred_element_type=jnp.float32)
    g0 = g0_ref[...]
    g1 = g1_ref[...]
    y0lo, y0hi = _unpack2(y0_ref[...])
    y1lo, y1hi = _unpack2(y1_ref[...])
    out_ref[:, :H] = g0 * y0lo + g1 * y1lo + sh[:, :H]
    out_ref[:, H:] = g0 * y0hi + g1 * y1hi + sh[:, H:]


def _combine(x, y0, y1, g0, g1, Ws1, Ws2, Ws3):
    row = lambda: pl.BlockSpec((BTC, D), lambda t: (t, 0))
    pkrow = lambda: pl.BlockSpec((BTC, H), lambda t: (t, 0))
    col = lambda: pl.BlockSpec((BTC, 1), lambda t: (t, 0))
    full = lambda shape: pl.BlockSpec(shape, lambda t: (0,) * len(shape))
    return pl.pallas_call(
        _combine_kernel,
        grid=(T // BTC,),
        in_specs=[pkrow(), pkrow(), pkrow(), col(), col(),
                  full((DF, D)), full((D, DF)), full((DF, D))],
        out_specs=row(),
        out_shape=jax.ShapeDtypeStruct((T, D), jnp.float32),
    )(x, y0, y1, g0, g1, Ws1, Ws2, Ws3)


def kernel(x, Wg, W1, W2, W3, Ws1, Ws2, Ws3):
    pos0, pos1, g0, g1, be, xpk = _route(x, Wg)
    pos0w = pos0.reshape(NW, NCH, CH)
    pos1w = pos1.reshape(NW, NCH, CH)
    xg = _sc_dispatch(xpk, pos0w, pos1w)
    ys = _ffn(be.reshape(NBLK), xg, W1, W2, W3)
    y0, y1 = _sc_gather(ys, pos0w, pos1w)
    return _combine(xpk, y0, y1, g0, g1, Ws1, Ws2, Ws3)


# speedup vs baseline: 1.4106x; 1.4106x over previous
import functools

import jax
import jax.numpy as jnp
from jax import lax
from jax.experimental import pallas as pl
from jax.experimental.pallas import tpu as pltpu
from jax.experimental.pallas import tpu_sc as plsc

E = 8
D = 1024
DF = 1024
T = 4096

BLK = 256
NBLK = 2 * T // BLK + E
NSLOT = NBLK * BLK

NW = 32
TPW = T // NW
CH = 64
NCH = TPW // CH

BTC = 256
H = D // 2


def _pack2(v):
    u = jax.lax.bitcast_convert_type(v, jnp.uint32)
    u = (u + jnp.uint32(0x8000)) >> 16
    a = u[:, :H]
    b = u[:, H:]
    return jax.lax.bitcast_convert_type((b << 16) | a, jnp.int32)


def _unpack2(p):
    u = jax.lax.bitcast_convert_type(p, jnp.uint32)
    lo = jax.lax.bitcast_convert_type(u << 16, jnp.float32)
    hi = jax.lax.bitcast_convert_type(u & jnp.uint32(0xFFFF0000), jnp.float32)
    return lo, hi


def _route_kernel(x_ref, wg_ref, pos0_ref, pos1_ref, g0_ref, g1_ref, be_ref,
                  xpk_ref):
    xb = x_ref[...]
    xpk_ref[...] = _pack2(xb)
    logits = jnp.dot(xb, wg_ref[...].T, preferred_element_type=jnp.float32)
    z = logits - jnp.max(logits, axis=-1, keepdims=True)
    ez = jnp.exp(z)
    sc = ez / jnp.sum(ez, axis=-1, keepdims=True)

    lane = lax.broadcasted_iota(jnp.int32, sc.shape, 1)
    m1 = jnp.max(sc, axis=-1, keepdims=True)
    i1 = jnp.min(jnp.where(sc == m1, lane, E), axis=-1, keepdims=True)
    s2 = jnp.where(lane == i1, -jnp.inf, sc)
    m2 = jnp.max(s2, axis=-1, keepdims=True)
    i2 = jnp.min(jnp.where(s2 == m2, lane, E), axis=-1, keepdims=True)

    C1 = (lane == i1).astype(jnp.float32)
    C2 = (lane == i2).astype(jnp.float32)

    R = 512
    G = T // R
    ri = lax.broadcasted_iota(jnp.int32, (R, R), 0)
    ci = lax.broadcasted_iota(jnp.int32, (R, R), 1)
    trilx = (ci < ri).astype(jnp.float32)
    ones_row = jnp.ones((1, R), jnp.float32)

    p1, p2, s1, s2l = [], [], [], []
    for g in range(G):
        cg1 = C1[g * R:(g + 1) * R, :]
        cg2 = C2[g * R:(g + 1) * R, :]
        p1.append(jnp.dot(trilx, cg1, preferred_element_type=jnp.float32))
        p2.append(jnp.dot(trilx, cg2, preferred_element_type=jnp.float32))
        s1.append(jnp.dot(ones_row, cg1, preferred_element_type=jnp.float32))
        s2l.append(jnp.dot(ones_row, cg2, preferred_element_type=jnp.float32))
    S1 = jnp.concatenate(s1, axis=0)
    S2 = jnp.concatenate(s2l, axis=0)
    gr = lax.broadcasted_iota(jnp.int32, (G, G), 0)
    gc = lax.broadcasted_iota(jnp.int32, (G, G), 1)
    trilg = (gc < gr).astype(jnp.float32)
    O1 = jnp.dot(trilg, S1, preferred_element_type=jnp.float32)
    O2 = jnp.dot(trilg, S2, preferred_element_type=jnp.float32)
    prefix1 = jnp.concatenate([p1[g] + O1[g:g + 1, :] for g in range(G)], 0)
    prefix2 = jnp.concatenate([p2[g] + O2[g:g + 1, :] for g in range(G)], 0)
    tot1 = jnp.sum(S1, axis=0, keepdims=True)
    tot2 = jnp.sum(S2, axis=0, keepdims=True)

    counts = tot1 + tot2
    nb = jnp.ceil(counts / BLK)
    er = lax.broadcasted_iota(jnp.int32, (E, E), 0)
    ec = lax.broadcasted_iota(jnp.int32, (E, E), 1)
    upx = (er < ec).astype(jnp.float32)
    slot_base = BLK * jnp.dot(nb, upx, preferred_element_type=jnp.float32)

    rank1 = jnp.sum(prefix1 * C1, axis=1, keepdims=True)
    base1 = jnp.sum(slot_base * C1, axis=1, keepdims=True)
    rank2 = jnp.sum((tot1 + prefix2) * C2, axis=1, keepdims=True)
    base2 = jnp.sum(slot_base * C2, axis=1, keepdims=True)
    pos0_ref[...] = (base1 + rank1).astype(jnp.int32)
    pos1_ref[...] = (base2 + rank2).astype(jnp.int32)
    g0_ref[...] = m1
    g1_ref[...] = m2

    cum_incl = jnp.sum((ec <= er).astype(jnp.float32) * nb, axis=1,
                       keepdims=True).astype(jnp.int32)
    bi = lax.broadcasted_iota(jnp.int32, (E, NBLK), 1)
    be = jnp.sum((bi >= cum_incl).astype(jnp.int32), axis=0, keepdims=True)
    be_ref[...] = be


def _route(x, Wg):
    full = lambda shape: pl.BlockSpec(shape, lambda: (0,) * len(shape))
    return pl.pallas_call(
        _route_kernel,
        in_specs=[full((T, D)), full((E, D))],
        out_specs=[full((T, 1)), full((T, 1)), full((T, 1)), full((T, 1)),
                   full((1, NBLK)), full((T, H))],
        out_shape=[
            jax.ShapeDtypeStruct((T, 1), jnp.int32),
            jax.ShapeDtypeStruct((T, 1), jnp.int32),
            jax.ShapeDtypeStruct((T, 1), jnp.float32),
            jax.ShapeDtypeStruct((T, 1), jnp.float32),
            jax.ShapeDtypeStruct((1, NBLK), jnp.int32),
            jax.ShapeDtypeStruct((T, H), jnp.int32),
        ],
    )(x, Wg)


@functools.cache
def _sc_mesh():
    return plsc.VectorSubcoreMesh(core_axis_name="c", subcore_axis_name="s")


@functools.cache
def _sc_dispatch_kernel():
    @functools.partial(
        pl.kernel,
        out_type=jax.ShapeDtypeStruct((NSLOT, H), jnp.int32),
        mesh=_sc_mesh(),
        scratch_types=[
            pltpu.VMEM((NCH, CH), jnp.int32),
            pltpu.VMEM((NCH, CH), jnp.int32),
            pltpu.VMEM((TPW, H), jnp.int32),
            pltpu.SemaphoreType.DMA,
        ],
    )
    def body(x_hbm, pos0_hbm, pos1_hbm, xg_hbm, idx0_v, idx1_v, xbuf_v, sem):
        wid = lax.axis_index("s") * 2 + lax.axis_index("c")
        base = wid * TPW
        pltpu.sync_copy(pos0_hbm.at[wid], idx0_v)
        pltpu.sync_copy(pos1_hbm.at[wid], idx1_v)
        pltpu.sync_copy(x_hbm.at[pl.ds(base, TPW)], xbuf_v)
        cps = []
        for j in range(NCH):
            rows = xbuf_v.at[pl.ds(j * CH, CH)]
            cps.append(pltpu.async_copy(rows, xg_hbm.at[idx0_v.at[j]], sem))
            cps.append(pltpu.async_copy(rows, xg_hbm.at[idx1_v.at[j]], sem))
        for cp in cps:
            cp.wait()

    return body


def _sc_dispatch(x, pos0w, pos1w):
    return _sc_dispatch_kernel()(x, pos0w, pos1w)


def _ffn_kernel(be_ref, xg_ref, w1_ref, w2_ref, w3_ref, out_ref):
    @pl.when(be_ref[pl.program_id(0)] < E)
    def _():
        lo, hi = _unpack2(xg_ref[...])
        lo = lo.astype(jnp.bfloat16)
        hi = hi.astype(jnp.bfloat16)
        w1 = w1_ref[0].astype(jnp.bfloat16)
        w3 = w3_ref[0].astype(jnp.bfloat16)
        w2 = w2_ref[0].astype(jnp.bfloat16)
        h1 = (jnp.dot(lo, w1[:, :H].T, preferred_element_type=jnp.float32)
              + jnp.dot(hi, w1[:, H:].T, preferred_element_type=jnp.float32))
        h3 = (jnp.dot(lo, w3[:, :H].T, preferred_element_type=jnp.float32)
              + jnp.dot(hi, w3[:, H:].T, preferred_element_type=jnp.float32))
        h = (jax.nn.silu(h1) * h3).astype(jnp.bfloat16)
        ye = jnp.dot(h, w2.T, preferred_element_type=jnp.float32)
        out_ref[...] = _pack2(ye)


def _ffn(be, xg, W1, W2, W3):
    wix = lambda b, be: (jnp.minimum(be[b], E - 1), 0, 0)
    grid_spec = pltpu.PrefetchScalarGridSpec(
        num_scalar_prefetch=1,
        grid=(NBLK,),
        in_specs=[
            pl.BlockSpec((BLK, H), lambda b, be: (b, 0)),
            pl.BlockSpec((1, DF, D), wix),
            pl.BlockSpec((1, D, DF), wix),
            pl.BlockSpec((1, DF, D), wix),
        ],
        out_specs=pl.BlockSpec((BLK, H), lambda b, be: (b, 0)),
    )
    return pl.pallas_call(
        _ffn_kernel,
        grid_spec=grid_spec,
        out_shape=jax.ShapeDtypeStruct((NSLOT, H), jnp.int32),
        compiler_params=pltpu.CompilerParams(
            dimension_semantics=("arbitrary",),
        ),
    )(be, xg, W1, W2, W3)


@functools.cache
def _sc_gather_kernel():
    @functools.partial(
        pl.kernel,
        out_type=[jax.ShapeDtypeStruct((T, H), jnp.int32),
                  jax.ShapeDtypeStruct((T, H), jnp.int32)],
        mesh=_sc_mesh(),
        scratch_types=[
            pltpu.VMEM((NCH, CH), jnp.int32),
            pltpu.VMEM((NCH, CH), jnp.int32),
            pltpu.VMEM((CH, H), jnp.int32),
            pltpu.VMEM((CH, H), jnp.int32),
            pltpu.SemaphoreType.DMA,
            pltpu.SemaphoreType.DMA,
        ],
    )
    def body(ys_hbm, pos0_hbm, pos1_hbm, y0_hbm, y1_hbm, idx0_v, idx1_v,
             b0_v, b1_v, s0, s1):
        wid = lax.axis_index("s") * 2 + lax.axis_index("c")
        base = wid * TPW
        pltpu.sync_copy(pos0_hbm.at[wid], idx0_v)
        pltpu.sync_copy(pos1_hbm.at[wid], idx1_v)
        a = pltpu.async_copy(ys_hbm.at[idx0_v.at[0]], b0_v, s0)
        b = pltpu.async_copy(ys_hbm.at[idx1_v.at[0]], b1_v, s1)
        for j in range(NCH):
            a.wait()
            b.wait()
            pltpu.sync_copy(b0_v, y0_hbm.at[pl.ds(base + j * CH, CH)])
            pltpu.sync_copy(b1_v, y1_hbm.at[pl.ds(base + j * CH, CH)])
            if j + 1 < NCH:
                a = pltpu.async_copy(ys_hbm.at[idx0_v.at[j + 1]], b0_v, s0)
                b = pltpu.async_copy(ys_hbm.at[idx1_v.at[j + 1]], b1_v, s1)

    return body


def _sc_gather(ys, pos0w, pos1w):
    return _sc_gather_kernel()(ys, pos0w, pos1w)


def _combine_kernel(x_ref, y0_ref, y1_ref, g0_ref, g1_ref,
                    ws1_ref, ws2_ref, ws3_ref, out_ref):
    lo, hi = _unpack2(x_ref[...])
    lo = lo.astype(jnp.bfloat16)
    hi = hi.astype(jnp.bfloat16)
    ws1 = ws1_ref[...].astype(jnp.bfloat16)
    ws3 = ws3_ref[...].astype(jnp.bfloat16)
    ws2 = ws2_ref[...].astype(jnp.bfloat16)
    h1 = (jnp.dot(lo, ws1[:, :H].T, preferred_element_type=jnp.float32)
          + jnp.dot(hi, ws1[:, H:].T, preferred_element_type=jnp.float32))
    h3 = (jnp.dot(lo, ws3[:, :H].T, preferred_element_type=jnp.float32)
          + jnp.dot(hi, ws3[:, H:].T, preferred_element_type=jnp.float32))
    h = (jax.nn.silu(h1) * h3).astype(jnp.bfloat16)
    sh = jnp.dot(h, ws2.T, preferred_element_type=jnp.float32)
    g0 = g0_ref[...]
    g1 = g1_ref[...]
    y0lo, y0hi = _unpack2(y0_ref[...])
    y1lo, y1hi = _unpack2(y1_ref[...])
    out_ref[:, :H] = g0 * y0lo + g1 * y1lo + sh[:, :H]
    out_ref[:, H:] = g0 * y0hi + g1 * y1hi + sh[:, H:]


def _combine(x, y0, y1, g0, g1, Ws1, Ws2, Ws3):
    row = lambda: pl.BlockSpec((BTC, D), lambda t: (t, 0))
    pkrow = lambda: pl.BlockSpec((BTC, H), lambda t: (t, 0))
    col = lambda: pl.BlockSpec((BTC, 1), lambda t: (t, 0))
    full = lambda shape: pl.BlockSpec(shape, lambda t: (0,) * len(shape))
    return pl.pallas_call(
        _combine_kernel,
        grid=(T // BTC,),
        in_specs=[pkrow(), pkrow(), pkrow(), col(), col(),
                  full((DF, D)), full((D, DF)), full((DF, D))],
        out_specs=row(),
        out_shape=jax.ShapeDtypeStruct((T, D), jnp.float32),
    )(x, y0, y1, g0, g1, Ws1, Ws2, Ws3)


def kernel(x, Wg, W1, W2, W3, Ws1, Ws2, Ws3):
    pos0, pos1, g0, g1, be, xpk = _route(x, Wg)
    pos0w = pos0.reshape(NW, NCH, CH)
    pos1w = pos1.reshape(NW, NCH, CH)
    xg = _sc_dispatch(xpk, pos0w, pos1w)
    ys = _ffn(be.reshape(NBLK), xg, W1, W2, W3)
    y0, y1 = _sc_gather(ys, pos0w, pos1w)
    return _combine(xpk, y0, y1, g0, g1, Ws1, Ws2, Ws3)
